# baseline (speedup 1.0000x reference)
import jax
import jax.numpy as jnp
from jax.experimental import pallas as pl
from jax.experimental.pallas import tpu as pltpu

_B, _T, _D = 16, 1024, 32
_K, _CD = 8192, 32
_M = 256
_NBLK = (_B * _T) // _M
_EPS = 1e-12


def _codebook_step(v_ref, g_ref, b_ref, fc_ref, cb_ref, c2_ref):
    v = v_ref[...]
    vn = jnp.sqrt(jnp.sum(v * v, axis=1, keepdims=True))
    w = g_ref[...].reshape(_D, 1) * v / vn
    cb = jnp.dot(fc_ref[...], w.T,
                 preferred_element_type=jnp.float32) + b_ref[...]
    cb_ref[...] = cb
    c2 = jnp.sum(cb * cb, axis=1).reshape(1, _K)
    c2_ref[...] = jnp.broadcast_to(c2, (_M, _K))


def _vq_step(z_ref, cb_ref, c2_ref, zq_ref, idx_ref, loss_ref):
    cb = cb_ref[...]
    z = z_ref[...]
    z2 = jnp.sum(z * z, axis=1, keepdims=True)
    cross = jnp.dot(z, cb.T, preferred_element_type=jnp.float32)
    d2 = z2 - 2.0 * cross + c2_ref[...]

    idx = jnp.argmin(d2, axis=1).astype(jnp.int32)
    idx_ref[...] = idx.reshape(1, 1, _M)

    kiota = jax.lax.broadcasted_iota(jnp.int32, (_M, _K), 1)
    onehot = (kiota == idx.reshape(_M, 1)).astype(jnp.float32)
    zq = jnp.dot(onehot, cb, preferred_element_type=jnp.float32)

    diff = z - zq
    loss_ref[...] = jnp.sum(diff * diff).reshape(1, 1, 1)

    norm_src = jnp.sqrt(jnp.sum(z * z, axis=1, keepdims=True))
    norm_tgt = jnp.sqrt(jnp.sum(zq * zq, axis=1, keepdims=True))
    u = z / jnp.maximum(norm_src, _EPS)
    q = zq / jnp.maximum(norm_tgt, _EPS)
    w_ = u + q
    wn = jnp.sqrt(jnp.sum(w_ * w_, axis=1, keepdims=True))
    w_ = w_ / jnp.maximum(wn, _EPS)
    rotated = (z
               - 2.0 * jnp.sum(z * w_, axis=1, keepdims=True) * w_
               + 2.0 * jnp.sum(z * u, axis=1, keepdims=True) * q)
    scale = norm_tgt / jnp.maximum(norm_src, _EPS)
    zq_ref[...] = rotated * scale


def kernel(z, v, g, b, frozen_codebook):
    zf = z.reshape(_B * _T, _D)
    g2 = g.reshape(1, _D)
    b2 = b.reshape(1, _D)

    cb, c2b = pl.pallas_call(
        _codebook_step,
        in_specs=[
            pl.BlockSpec((_D, _CD), lambda: (0, 0)),
            pl.BlockSpec((1, _D), lambda: (0, 0)),
            pl.BlockSpec((1, _D), lambda: (0, 0)),
            pl.BlockSpec((_K, _CD), lambda: (0, 0)),
        ],
        out_specs=[
            pl.BlockSpec((_K, _D), lambda: (0, 0)),
            pl.BlockSpec((_M, _K), lambda: (0, 0)),
        ],
        out_shape=[
            jax.ShapeDtypeStruct((_K, _D), jnp.float32),
            jax.ShapeDtypeStruct((_M, _K), jnp.float32),
        ],
    )(v, g2, b2, frozen_codebook)

    zq, idx, lparts = pl.pallas_call(
        _vq_step,
        grid=(_NBLK,),
        in_specs=[
            pl.BlockSpec((_M, _D), lambda i: (i, 0)),
            pl.BlockSpec((_K, _D), lambda i: (0, 0)),
            pl.BlockSpec((_M, _K), lambda i: (0, 0)),
        ],
        out_specs=[
            pl.BlockSpec((_M, _D), lambda i: (i, 0)),
            pl.BlockSpec((1, 1, _M), lambda i: (i, 0, 0)),
            pl.BlockSpec((1, 1, 1), lambda i: (i, 0, 0)),
        ],
        out_shape=[
            jax.ShapeDtypeStruct((_B * _T, _D), jnp.float32),
            jax.ShapeDtypeStruct((_NBLK, 1, _M), jnp.int32),
            jax.ShapeDtypeStruct((_NBLK, 1, 1), jnp.float32),
        ],
        compiler_params=pltpu.CompilerParams(
            dimension_semantics=("parallel",),
        ),
    )(zf, cb, c2b)

    z_q = zq.reshape(_B, _T, _D)
    indices = idx.reshape(_B, _T)
    commit_loss = jnp.sum(lparts) * (1.25 / (_B * _T * _D))
    return (z_q, indices, commit_loss)


# speedup vs baseline: 1.4596x; 1.4596x over previous
import functools

import jax
import jax.numpy as jnp
from jax import lax
from jax.experimental import pallas as pl
from jax.experimental.pallas import tpu as pltpu
from jax.experimental.pallas import tpu_sc as plsc

_B, _T, _D = 16, 1024, 32
_K, _CD = 8192, 32
_N = _B * _T
_M = 256
_NBLK = _N // _M
_R = 4096
_NRBLK = _N // _R
_EPS = 1e-12

_NC, _NS = 2, 16
_NW = _NC * _NS
_BPW = _N // _NW


def _codebook_step(v_ref, g_ref, b_ref, fc_ref, cb_ref, c2_ref, cbp_ref):
    v = v_ref[...]
    vn = jnp.sqrt(jnp.sum(v * v, axis=1, keepdims=True))
    w = g_ref[...].reshape(_D, 1) * v / vn
    cb = jnp.dot(fc_ref[...], w.T,
                 preferred_element_type=jnp.float32) + b_ref[...]
    cb_ref[...] = cb
    c2 = jnp.sum(cb * cb, axis=1).reshape(1, _K)
    c2_ref[...] = jnp.broadcast_to(c2, (_M, _K))
    cbp_ref[...] = jnp.pad(cb, ((0, 0), (0, 128 - _D)))


def _argmin_step(z_ref, cb_ref, c2_ref, idx_ref):
    cb = cb_ref[...]
    z = z_ref[...]
    z2 = jnp.sum(z * z, axis=1, keepdims=True)
    cross = jnp.dot(z, cb.T, preferred_element_type=jnp.float32)
    d2 = z2 - 2.0 * cross + c2_ref[...]
    idx = jnp.argmin(d2, axis=1).astype(jnp.int32)
    idx_ref[...] = idx.reshape(1, 1, _M)


def _gather_sc(cbp_hbm, idx_hbm, out_hbm, idx_v, rows_v, sem):
    wid = lax.axis_index("s") * _NC + lax.axis_index("c")
    base = wid * _BPW
    pltpu.sync_copy(idx_hbm.at[pl.ds(base, _BPW)], idx_v)
    pltpu.async_copy(cbp_hbm.at[idx_v], rows_v, sem).wait()
    pltpu.sync_copy(rows_v, out_hbm.at[pl.ds(base, _BPW)])


def _rotate_step(z_ref, zq_ref, out_ref, loss_ref):
    z = z_ref[...]
    zq = zq_ref[:, :_D]
    diff = z - zq
    loss_ref[...] = jnp.sum(diff * diff).reshape(1, 1, 1)

    norm_src = jnp.sqrt(jnp.sum(z * z, axis=1, keepdims=True))
    norm_tgt = jnp.sqrt(jnp.sum(zq * zq, axis=1, keepdims=True))
    u = z / jnp.maximum(norm_src, _EPS)
    q = zq / jnp.maximum(norm_tgt, _EPS)
    w_ = u + q
    wn = jnp.sqrt(jnp.sum(w_ * w_, axis=1, keepdims=True))
    w_ = w_ / jnp.maximum(wn, _EPS)
    rotated = (z
               - 2.0 * jnp.sum(z * w_, axis=1, keepdims=True) * w_
               + 2.0 * jnp.sum(z * u, axis=1, keepdims=True) * q)
    scale = norm_tgt / jnp.maximum(norm_src, _EPS)
    out_ref[...] = rotated * scale


def kernel(z, v, g, b, frozen_codebook):
    zf = z.reshape(_N, _D)
    g2 = g.reshape(1, _D)
    b2 = b.reshape(1, _D)

    cb, c2b, cbp = pl.pallas_call(
        _codebook_step,
        in_specs=[
            pl.BlockSpec((_D, _CD), lambda: (0, 0)),
            pl.BlockSpec((1, _D), lambda: (0, 0)),
            pl.BlockSpec((1, _D), lambda: (0, 0)),
            pl.BlockSpec((_K, _CD), lambda: (0, 0)),
        ],
        out_specs=[
            pl.BlockSpec((_K, _D), lambda: (0, 0)),
            pl.BlockSpec((_M, _K), lambda: (0, 0)),
            pl.BlockSpec((_K, 128), lambda: (0, 0)),
        ],
        out_shape=[
            jax.ShapeDtypeStruct((_K, _D), jnp.float32),
            jax.ShapeDtypeStruct((_M, _K), jnp.float32),
            jax.ShapeDtypeStruct((_K, 128), jnp.float32),
        ],
    )(v, g2, b2, frozen_codebook)

    idx3 = pl.pallas_call(
        _argmin_step,
        grid=(_NBLK,),
        in_specs=[
            pl.BlockSpec((_M, _D), lambda i: (i, 0)),
            pl.BlockSpec((_K, _D), lambda i: (0, 0)),
            pl.BlockSpec((_M, _K), lambda i: (0, 0)),
        ],
        out_specs=pl.BlockSpec((1, 1, _M), lambda i: (i, 0, 0)),
        out_shape=jax.ShapeDtypeStruct((_NBLK, 1, _M), jnp.int32),
        compiler_params=pltpu.CompilerParams(
            dimension_semantics=("parallel",),
        ),
    )(zf, cb, c2b)
    idx = idx3.reshape(_N)

    gather = functools.partial(
        pl.kernel,
        mesh=plsc.VectorSubcoreMesh(core_axis_name="c", subcore_axis_name="s"),
        out_type=jax.ShapeDtypeStruct((_N, 128), jnp.float32),
        scratch_types=[
            pltpu.VMEM((_BPW,), jnp.int32),
            pltpu.VMEM((_BPW, 128), jnp.float32),
            pltpu.SemaphoreType.DMA,
        ],
    )(_gather_sc)
    zq_raw = gather(cbp, idx)

    zq, lparts = pl.pallas_call(
        _rotate_step,
        grid=(_NRBLK,),
        in_specs=[
            pl.BlockSpec((_R, _D), lambda i: (i, 0)),
            pl.BlockSpec((_R, 128), lambda i: (i, 0)),
        ],
        out_specs=[
            pl.BlockSpec((_R, _D), lambda i: (i, 0)),
            pl.BlockSpec((1, 1, 1), lambda i: (i, 0, 0)),
        ],
        out_shape=[
            jax.ShapeDtypeStruct((_N, _D), jnp.float32),
            jax.ShapeDtypeStruct((_NRBLK, 1, 1), jnp.float32),
        ],
        compiler_params=pltpu.CompilerParams(
            dimension_semantics=("parallel",),
        ),
    )(zf, zq_raw)

    z_q = zq.reshape(_B, _T, _D)
    indices = idx.reshape(_B, _T)
    commit_loss = jnp.sum(lparts) * (1.25 / (_N * _D))
    return (z_q, indices, commit_loss)


# speedup vs baseline: 1.6628x; 1.1393x over previous
import functools

import jax
import jax.numpy as jnp
from jax import lax
from jax.experimental import pallas as pl
from jax.experimental.pallas import tpu as pltpu
from jax.experimental.pallas import tpu_sc as plsc

_B, _T, _D = 16, 1024, 32
_K, _CD = 8192, 32
_N = _B * _T
_M = 512
_NBLK = _N // _M
_R = 4096
_NRBLK = _N // _R
_EPS = 1e-12

_NC, _NS = 2, 16
_NW = _NC * _NS
_BPW = _N // _NW


def _codebook_step(v_ref, g_ref, b_ref, fc_ref, cb_ref, c2_ref, cbp_ref):
    v = v_ref[...]
    vn = jnp.sqrt(jnp.sum(v * v, axis=1, keepdims=True))
    w = g_ref[...].reshape(_D, 1) * v / vn
    cb = jnp.dot(fc_ref[...], w.T,
                 preferred_element_type=jnp.float32) + b_ref[...]
    cb_ref[...] = cb
    c2 = jnp.sum(cb * cb, axis=1).reshape(1, _K)
    c2_ref[...] = jnp.broadcast_to(c2, (_M, _K))
    cbp_ref[...] = jnp.pad(cb, ((0, 0), (0, 128 - _D)))


def _argmin_step(z_ref, cb_ref, c2_ref, idx_ref):
    cb = cb_ref[...]
    z = z_ref[...]
    z2 = jnp.sum(z * z, axis=1, keepdims=True)
    ncross = jnp.dot(-2.0 * z, cb.T,
                     preferred_element_type=jnp.float32)
    d2 = z2 + ncross + c2_ref[...]
    idx = jnp.argmin(d2, axis=1).astype(jnp.int32)
    idx_ref[...] = idx.reshape(1, 1, _M)


def _gather_sc(cbp_hbm, idx_hbm, out_hbm, idx_v, rows_v, sem):
    wid = lax.axis_index("s") * _NC + lax.axis_index("c")
    base = wid * _BPW
    pltpu.sync_copy(idx_hbm.at[pl.ds(base, _BPW)], idx_v)
    pltpu.async_copy(cbp_hbm.at[idx_v], rows_v, sem).wait()
    pltpu.sync_copy(rows_v, out_hbm.at[pl.ds(base, _BPW)])


def _rotate_step(z_ref, zq_ref, out_ref, loss_ref):
    z = z_ref[...]
    zq = zq_ref[:, :_D]
    diff = z - zq
    loss_ref[...] = jnp.sum(diff * diff).reshape(1, 1, 1)

    norm_src = jnp.sqrt(jnp.sum(z * z, axis=1, keepdims=True))
    norm_tgt = jnp.sqrt(jnp.sum(zq * zq, axis=1, keepdims=True))
    u = z / jnp.maximum(norm_src, _EPS)
    q = zq / jnp.maximum(norm_tgt, _EPS)
    w_ = u + q
    wn = jnp.sqrt(jnp.sum(w_ * w_, axis=1, keepdims=True))
    w_ = w_ / jnp.maximum(wn, _EPS)
    rotated = (z
               - 2.0 * jnp.sum(z * w_, axis=1, keepdims=True) * w_
               + 2.0 * jnp.sum(z * u, axis=1, keepdims=True) * q)
    scale = norm_tgt / jnp.maximum(norm_src, _EPS)
    out_ref[...] = rotated * scale


def kernel(z, v, g, b, frozen_codebook):
    zf = z.reshape(_N, _D)
    g2 = g.reshape(1, _D)
    b2 = b.reshape(1, _D)

    cb, c2b, cbp = pl.pallas_call(
        _codebook_step,
        in_specs=[
            pl.BlockSpec((_D, _CD), lambda: (0, 0)),
            pl.BlockSpec((1, _D), lambda: (0, 0)),
            pl.BlockSpec((1, _D), lambda: (0, 0)),
            pl.BlockSpec((_K, _CD), lambda: (0, 0)),
        ],
        out_specs=[
            pl.BlockSpec((_K, _D), lambda: (0, 0)),
            pl.BlockSpec((_M, _K), lambda: (0, 0)),
            pl.BlockSpec((_K, 128), lambda: (0, 0)),
        ],
        out_shape=[
            jax.ShapeDtypeStruct((_K, _D), jnp.float32),
            jax.ShapeDtypeStruct((_M, _K), jnp.float32),
            jax.ShapeDtypeStruct((_K, 128), jnp.float32),
        ],
    )(v, g2, b2, frozen_codebook)

    idx3 = pl.pallas_call(
        _argmin_step,
        grid=(_NBLK,),
        in_specs=[
            pl.BlockSpec((_M, _D), lambda i: (i, 0)),
            pl.BlockSpec((_K, _D), lambda i: (0, 0)),
            pl.BlockSpec((_M, _K), lambda i: (0, 0)),
        ],
        out_specs=pl.BlockSpec((1, 1, _M), lambda i: (i, 0, 0)),
        out_shape=jax.ShapeDtypeStruct((_NBLK, 1, _M), jnp.int32),
        compiler_params=pltpu.CompilerParams(
            dimension_semantics=("parallel",),
        ),
    )(zf, cb, c2b)
    idx = idx3.reshape(_N)

    gather = functools.partial(
        pl.kernel,
        mesh=plsc.VectorSubcoreMesh(core_axis_name="c", subcore_axis_name="s"),
        out_type=jax.ShapeDtypeStruct((_N, 128), jnp.float32),
        scratch_types=[
            pltpu.VMEM((_BPW,), jnp.int32),
            pltpu.VMEM((_BPW, 128), jnp.float32),
            pltpu.SemaphoreType.DMA,
        ],
    )(_gather_sc)
    zq_raw = gather(cbp, idx)

    zq, lparts = pl.pallas_call(
        _rotate_step,
        grid=(_NRBLK,),
        in_specs=[
            pl.BlockSpec((_R, _D), lambda i: (i, 0)),
            pl.BlockSpec((_R, 128), lambda i: (i, 0)),
        ],
        out_specs=[
            pl.BlockSpec((_R, _D), lambda i: (i, 0)),
            pl.BlockSpec((1, 1, 1), lambda i: (i, 0, 0)),
        ],
        out_shape=[
            jax.ShapeDtypeStruct((_N, _D), jnp.float32),
            jax.ShapeDtypeStruct((_NRBLK, 1, 1), jnp.float32),
        ],
        compiler_params=pltpu.CompilerParams(
            dimension_semantics=("parallel",),
        ),
    )(zf, zq_raw)

    z_q = zq.reshape(_B, _T, _D)
    indices = idx.reshape(_B, _T)
    commit_loss = jnp.sum(lparts) * (1.25 / (_N * _D))
    return (z_q, indices, commit_loss)


# speedup vs baseline: 1.7654x; 1.0617x over previous
import functools

import jax
import jax.numpy as jnp
from jax import lax
from jax.experimental import pallas as pl
from jax.experimental.pallas import tpu as pltpu
from jax.experimental.pallas import tpu_sc as plsc

_B, _T, _D = 16, 1024, 32
_K, _CD = 8192, 32
_N = _B * _T
_M = 512
_NBLK = _N // _M
_R = 8192
_NRBLK = _N // _R
_EPS = 1e-12

_NC, _NS = 2, 16
_NW = _NC * _NS
_BPW = _N // _NW


def _argmin_step(z_ref, v_ref, g_ref, b_ref, fc_ref,
                 idx_ref, cbp_ref, cb_ref, c2_ref):
    i = pl.program_id(0)

    @pl.when(i == 0)
    def _init():
        v = v_ref[...]
        vn = jnp.sqrt(jnp.sum(v * v, axis=1, keepdims=True))
        w = g_ref[...].reshape(_D, 1) * v / vn
        cb = jnp.dot(fc_ref[...], w.T,
                     preferred_element_type=jnp.float32) + b_ref[...]
        cb_ref[...] = cb
        c2 = jnp.sum(cb * cb, axis=1).reshape(1, _K)
        c2_ref[...] = jnp.broadcast_to(c2, (_M, _K))
        cbp_ref[...] = jnp.pad(cb, ((0, 0), (0, 128 - _D)))

    cb = cb_ref[...]
    z = z_ref[...]
    z2 = jnp.sum(z * z, axis=1, keepdims=True)
    ncross = jnp.dot(-2.0 * z, cb.T,
                     preferred_element_type=jnp.float32)
    d2 = z2 + ncross + c2_ref[...]
    idx = jnp.argmin(d2, axis=1).astype(jnp.int32)
    idx_ref[...] = idx.reshape(1, 1, _M)


def _gather_sc(cbp_hbm, idx_hbm, out_hbm, idx_v, rows_v, sem):
    wid = lax.axis_index("s") * _NC + lax.axis_index("c")
    base = wid * _BPW
    pltpu.sync_copy(idx_hbm.at[pl.ds(base, _BPW)], idx_v)
    pltpu.async_copy(cbp_hbm.at[idx_v], rows_v, sem).wait()
    pltpu.sync_copy(rows_v, out_hbm.at[pl.ds(base, _BPW)])


def _rotate_step(z_ref, zq_ref, out_ref, loss_ref):
    z = z_ref[...]
    zq = zq_ref[:, :_D]
    diff = z - zq
    loss_ref[...] = jnp.sum(diff * diff).reshape(1, 1, 1)

    norm_src = jnp.sqrt(jnp.sum(z * z, axis=1, keepdims=True))
    norm_tgt = jnp.sqrt(jnp.sum(zq * zq, axis=1, keepdims=True))
    u = z / jnp.maximum(norm_src, _EPS)
    q = zq / jnp.maximum(norm_tgt, _EPS)
    w_ = u + q
    wn = jnp.sqrt(jnp.sum(w_ * w_, axis=1, keepdims=True))
    w_ = w_ / jnp.maximum(wn, _EPS)
    rotated = (z
               - 2.0 * jnp.sum(z * w_, axis=1, keepdims=True) * w_
               + 2.0 * jnp.sum(z * u, axis=1, keepdims=True) * q)
    scale = norm_tgt / jnp.maximum(norm_src, _EPS)
    out_ref[...] = rotated * scale


def kernel(z, v, g, b, frozen_codebook):
    zf = z.reshape(_N, _D)
    g2 = g.reshape(1, _D)
    b2 = b.reshape(1, _D)

    idx3, cbp = pl.pallas_call(
        _argmin_step,
        grid=(_NBLK,),
        in_specs=[
            pl.BlockSpec((_M, _D), lambda i: (i, 0)),
            pl.BlockSpec((_D, _CD), lambda i: (0, 0)),
            pl.BlockSpec((1, _D), lambda i: (0, 0)),
            pl.BlockSpec((1, _D), lambda i: (0, 0)),
            pl.BlockSpec((_K, _CD), lambda i: (0, 0)),
        ],
        out_specs=[
            pl.BlockSpec((1, 1, _M), lambda i: (i, 0, 0)),
            pl.BlockSpec((_K, 128), lambda i: (0, 0)),
        ],
        out_shape=[
            jax.ShapeDtypeStruct((_NBLK, 1, _M), jnp.int32),
            jax.ShapeDtypeStruct((_K, 128), jnp.float32),
        ],
        scratch_shapes=[
            pltpu.VMEM((_K, _D), jnp.float32),
            pltpu.VMEM((_M, _K), jnp.float32),
        ],
        compiler_params=pltpu.CompilerParams(
            dimension_semantics=("arbitrary",),
        ),
    )(zf, v, g2, b2, frozen_codebook)
    idx = idx3.reshape(_N)

    gather = functools.partial(
        pl.kernel,
        mesh=plsc.VectorSubcoreMesh(core_axis_name="c", subcore_axis_name="s"),
        out_type=jax.ShapeDtypeStruct((_N, 128), jnp.float32),
        scratch_types=[
            pltpu.VMEM((_BPW,), jnp.int32),
            pltpu.VMEM((_BPW, 128), jnp.float32),
            pltpu.SemaphoreType.DMA,
        ],
    )(_gather_sc)
    zq_raw = gather(cbp, idx)

    zq, lparts = pl.pallas_call(
        _rotate_step,
        grid=(_NRBLK,),
        in_specs=[
            pl.BlockSpec((_R, _D), lambda i: (i, 0)),
            pl.BlockSpec((_R, 128), lambda i: (i, 0)),
        ],
        out_specs=[
            pl.BlockSpec((_R, _D), lambda i: (i, 0)),
            pl.BlockSpec((1, 1, 1), lambda i: (i, 0, 0)),
        ],
        out_shape=[
            jax.ShapeDtypeStruct((_N, _D), jnp.float32),
            jax.ShapeDtypeStruct((_NRBLK, 1, 1), jnp.float32),
        ],
        compiler_params=pltpu.CompilerParams(
            dimension_semantics=("parallel",),
        ),
    )(zf, zq_raw)

    z_q = zq.reshape(_B, _T, _D)
    indices = idx.reshape(_B, _T)
    commit_loss = jnp.sum(lparts) * (1.25 / (_N * _D))
    return (z_q, indices, commit_loss)
